# baseline (speedup 1.0000x reference)
import functools

import jax
import jax.numpy as jnp
from jax.experimental import pallas as pl
from jax.experimental.pallas import tpu as pltpu

D = 1024
H = 2048
HALF = D // 2


def _flow_body(x_ref, w1_hbm, b1_ref, w2_hbm, b2_ref, permc_ref, perm_ref,
               y_ref, ld_ref, w1_v, w2_v, g1_v, m_v, sem1, sem2):
    i = pl.program_id(0)
    cp1 = pltpu.make_async_copy(w1_hbm, w1_v, sem1)
    cp2 = pltpu.make_async_copy(w2_hbm, w2_v, sem2)

    @pl.when(i == 0)
    def _start():
        cp1.start()
        cp2.start()
        pc = permc_ref[...]
        cols = jax.lax.broadcasted_iota(jnp.int32, (D, HALF), 1)
        g1_v[...] = (pc == cols).astype(jnp.float32)
        pm = perm_ref[...]
        jrows = jax.lax.broadcasted_iota(jnp.int32, (HALF, D), 0)
        m_v[...] = (pm == jrows + HALF).astype(jnp.float32)

    xb = x_ref[...]
    x1 = jnp.dot(xb, g1_v[...], preferred_element_type=jnp.float32)

    @pl.when(i == 0)
    def _wait1():
        cp1.wait()

    h = jnp.tanh(jnp.dot(x1, w1_v[...],
                         preferred_element_type=jnp.float32) + b1_ref[...])

    @pl.when(i == 0)
    def _wait2():
        cp2.wait()

    params = jnp.dot(h, w2_v[...],
                     preferred_element_type=jnp.float32) + b2_ref[...]
    s = jnp.tanh(params[:, :HALF])
    t = params[:, HALF:]

    M = m_v[...]
    S = jnp.dot(s, M, preferred_element_type=jnp.float32)
    T = jnp.dot(t, M, preferred_element_type=jnp.float32)

    y_ref[...] = xb * jnp.exp(S) + T
    ld_ref[...] = jnp.sum(s, axis=1, keepdims=True)


@functools.partial(jax.jit, static_argnames=("interpret",))
def _run(x, W1, b1, W2, b2, perm_col, perm_2d, interpret=False):
    N = x.shape[0]
    R = 512
    grid = (N // R,)

    y, ld = pl.pallas_call(
        _flow_body,
        grid=grid,
        in_specs=[
            pl.BlockSpec((R, D), lambda i: (i, 0)),
            pl.BlockSpec(memory_space=pltpu.HBM),
            pl.BlockSpec((1, H), lambda i: (0, 0)),
            pl.BlockSpec(memory_space=pltpu.HBM),
            pl.BlockSpec((1, D), lambda i: (0, 0)),
            pl.BlockSpec((D, 1), lambda i: (0, 0)),
            pl.BlockSpec((1, D), lambda i: (0, 0)),
        ],
        out_specs=[
            pl.BlockSpec((R, D), lambda i: (i, 0)),
            pl.BlockSpec((R, 1), lambda i: (i, 0)),
        ],
        out_shape=[
            jax.ShapeDtypeStruct((N, D), jnp.float32),
            jax.ShapeDtypeStruct((N, 1), jnp.float32),
        ],
        scratch_shapes=[
            pltpu.VMEM((HALF, H), jnp.float32),
            pltpu.VMEM((H, D), jnp.float32),
            pltpu.VMEM((D, HALF), jnp.float32),
            pltpu.VMEM((HALF, D), jnp.float32),
            pltpu.SemaphoreType.DMA,
            pltpu.SemaphoreType.DMA,
        ],
        interpret=interpret,
    )(x, W1, b1.reshape(1, H), W2, b2.reshape(1, D), perm_col, perm_2d)
    return y, ld[:, 0]


def kernel(x, W1, b1, W2, b2, perm):
    perm = perm.astype(jnp.int32)
    return _run(x, W1, b1, W2, b2, perm.reshape(D, 1), perm.reshape(1, D))


# speedup vs baseline: 1.0385x; 1.0385x over previous
import functools

import jax
import jax.numpy as jnp
from jax.experimental import pallas as pl
from jax.experimental.pallas import tpu as pltpu

D = 1024
H = 2048
HALF = D // 2


def _flow_body(x_ref, w1_hbm, b1_ref, w2_hbm, b2_ref, permc_ref, perm_ref,
               y_ref, ld_ref, w1_v, w2_v, g1_v, m_v, pass_v, sem1, sem2):
    i = pl.program_id(0)
    cp1 = pltpu.make_async_copy(w1_hbm, w1_v, sem1)
    cp2 = pltpu.make_async_copy(w2_hbm, w2_v, sem2)

    @pl.when(i == 0)
    def _start():
        cp1.start()
        cp2.start()
        pc = permc_ref[...]
        cols = jax.lax.broadcasted_iota(jnp.int32, (D, HALF), 1)
        g1_v[...] = (pc == cols).astype(jnp.float32)
        pm = perm_ref[...]
        jrows = jax.lax.broadcasted_iota(jnp.int32, (HALF, D), 0)
        m_v[...] = (pm == jrows + HALF).astype(jnp.float32)
        pass_v[...] = (pm < HALF).astype(jnp.float32)

    xb = x_ref[...]
    x1 = jnp.dot(xb, g1_v[...], preferred_element_type=jnp.float32)

    @pl.when(i == 0)
    def _wait1():
        cp1.wait()

    h = jnp.tanh(jnp.dot(x1, w1_v[...],
                         preferred_element_type=jnp.float32) + b1_ref[...])

    @pl.when(i == 0)
    def _wait2():
        cp2.wait()

    params = jnp.dot(h, w2_v[...],
                     preferred_element_type=jnp.float32) + b2_ref[...]
    s = jnp.tanh(params[:, :HALF])
    t = params[:, HALF:]

    M = m_v[...]
    es = jnp.exp(s)
    E = jnp.dot(es, M, preferred_element_type=jnp.float32) + pass_v[...]
    T = jnp.dot(t, M, preferred_element_type=jnp.float32)

    y_ref[...] = xb * E + T
    ld_ref[...] = jnp.sum(s, axis=1, keepdims=True)


@functools.partial(jax.jit, static_argnames=("interpret",))
def _run(x, W1, b1, W2, b2, perm_col, perm_2d, interpret=False):
    N = x.shape[0]
    R = 1024
    grid = (N // R,)

    y, ld = pl.pallas_call(
        _flow_body,
        grid=grid,
        in_specs=[
            pl.BlockSpec((R, D), lambda i: (i, 0)),
            pl.BlockSpec(memory_space=pltpu.HBM),
            pl.BlockSpec((1, H), lambda i: (0, 0)),
            pl.BlockSpec(memory_space=pltpu.HBM),
            pl.BlockSpec((1, D), lambda i: (0, 0)),
            pl.BlockSpec((D, 1), lambda i: (0, 0)),
            pl.BlockSpec((1, D), lambda i: (0, 0)),
        ],
        out_specs=[
            pl.BlockSpec((R, D), lambda i: (i, 0)),
            pl.BlockSpec((R, 1), lambda i: (i, 0)),
        ],
        out_shape=[
            jax.ShapeDtypeStruct((N, D), jnp.float32),
            jax.ShapeDtypeStruct((N, 1), jnp.float32),
        ],
        scratch_shapes=[
            pltpu.VMEM((HALF, H), jnp.float32),
            pltpu.VMEM((H, D), jnp.float32),
            pltpu.VMEM((D, HALF), jnp.float32),
            pltpu.VMEM((HALF, D), jnp.float32),
            pltpu.VMEM((1, D), jnp.float32),
            pltpu.SemaphoreType.DMA,
            pltpu.SemaphoreType.DMA,
        ],
        interpret=interpret,
    )(x, W1, b1.reshape(1, H), W2, b2.reshape(1, D), perm_col, perm_2d)
    return y, ld.reshape(N)


def kernel(x, W1, b1, W2, b2, perm):
    perm = perm.astype(jnp.int32)
    return _run(x, W1, b1, W2, b2, perm.reshape(D, 1), perm.reshape(1, D))


# speedup vs baseline: 1.0594x; 1.0201x over previous
import functools

import jax
import jax.numpy as jnp
from jax.experimental import pallas as pl
from jax.experimental.pallas import tpu as pltpu

D = 1024
H = 2048
HALF = D // 2


def _flow_body(x_ref, w1_ref, b1_ref, w2_ref, b2_ref, permc_ref, perm_ref,
               y_ref, ld_ref):
    xb = x_ref[...]

    pc = permc_ref[...]
    cols = jax.lax.broadcasted_iota(jnp.int32, (D, HALF), 1)
    G1 = (pc == cols).astype(jnp.float32)

    x1 = jnp.dot(xb, G1, preferred_element_type=jnp.float32)
    h = jnp.tanh(jnp.dot(x1, w1_ref[...],
                         preferred_element_type=jnp.float32) + b1_ref[...])
    params = jnp.dot(h, w2_ref[...],
                     preferred_element_type=jnp.float32) + b2_ref[...]
    s = jnp.tanh(params[:, :HALF])
    t = params[:, HALF:]

    pm = perm_ref[...]
    jrows = jax.lax.broadcasted_iota(jnp.int32, (HALF, D), 0)
    M = (pm == jrows + HALF).astype(jnp.float32)

    S = jnp.dot(s, M, preferred_element_type=jnp.float32)
    T = jnp.dot(t, M, preferred_element_type=jnp.float32)

    y_ref[...] = xb * jnp.exp(S) + T
    ld_ref[...] = jnp.sum(s, axis=1, keepdims=True)


@functools.partial(jax.jit, static_argnames=("interpret",))
def _run(x, W1, b1, W2, b2, perm_col, perm_2d, interpret=False):
    N = x.shape[0]
    R = 1024
    grid = (N // R,)

    y, ld = pl.pallas_call(
        _flow_body,
        grid=grid,
        in_specs=[
            pl.BlockSpec((R, D), lambda i: (i, 0)),
            pl.BlockSpec((HALF, H), lambda i: (0, 0)),
            pl.BlockSpec((1, H), lambda i: (0, 0)),
            pl.BlockSpec((H, D), lambda i: (0, 0)),
            pl.BlockSpec((1, D), lambda i: (0, 0)),
            pl.BlockSpec((D, 1), lambda i: (0, 0)),
            pl.BlockSpec((1, D), lambda i: (0, 0)),
        ],
        out_specs=[
            pl.BlockSpec((R, D), lambda i: (i, 0)),
            pl.BlockSpec((R, 1), lambda i: (i, 0)),
        ],
        out_shape=[
            jax.ShapeDtypeStruct((N, D), jnp.float32),
            jax.ShapeDtypeStruct((N, 1), jnp.float32),
        ],
        interpret=interpret,
    )(x, W1, b1.reshape(1, H), W2, b2.reshape(1, D), perm_col, perm_2d)
    return y, ld[:, 0]


def kernel(x, W1, b1, W2, b2, perm):
    perm = perm.astype(jnp.int32)
    return _run(x, W1, b1, W2, b2, perm.reshape(D, 1), perm.reshape(1, D))
